# baseline (speedup 1.0000x reference)
import functools

import jax
import jax.numpy as jnp
from jax import lax
from jax.experimental import pallas as pl
from jax.experimental.pallas import tpu as pltpu
from jax.experimental.pallas import tpu_sc as plsc

B = 4096
INPUT_DIM = 768
HIDDEN_DIM = 512
LATENT_DIM = 32
NUM_EMBEDDINGS = 1024
BETA = 0.25

BM = 512
NB = B // BM

SC_NC = 1
SC_NS = 16
SC_NW = SC_NC * SC_NS
B_PER_W = B // SC_NW

GATHER_D = 128


def _fused_body(x_ref, w1_ref, b1_ref, w2_ref, b2_ref, cbt_ref, cb_ref,
                dw1_ref, db1_ref, dw2_ref, db2_ref,
                ze_ref, idx_ref, xr_ref, loss_ref):
    h = jnp.maximum(
        jnp.dot(x_ref[...], w1_ref[...], preferred_element_type=jnp.float32)
        + b1_ref[...], 0.0)
    z_e = (jnp.dot(h, w2_ref[...], preferred_element_type=jnp.float32)
           + b2_ref[...])
    ze_ref[...] = z_e

    cbt = cbt_ref[...]
    cnorm2 = jnp.sum(cbt * cbt, axis=0, keepdims=True)
    scores = jnp.dot(z_e, cbt, preferred_element_type=jnp.float32,
                     precision=lax.Precision.HIGHEST)
    d2 = cnorm2 - 2.0 * scores
    dmin = jnp.min(d2, axis=1, keepdims=True)
    iota = lax.broadcasted_iota(jnp.int32, d2.shape, 1)
    cand = jnp.where(d2 == dmin, iota, NUM_EMBEDDINGS)
    idx = jnp.min(cand, axis=1, keepdims=True)
    idx_ref[0] = idx

    onehot = (iota == idx).astype(jnp.float32)
    z_q = jnp.dot(onehot, cb_ref[...], preferred_element_type=jnp.float32,
                  precision=lax.Precision.HIGHEST)
    z_st = z_e + (z_q - z_e)
    diff = z_q - z_e
    part = jnp.sum(diff * diff, keepdims=True)

    @pl.when(pl.program_id(0) == 0)
    def _():
        loss_ref[...] = jnp.zeros_like(loss_ref)

    loss_ref[...] += part * ((1.0 + BETA) / (B * LATENT_DIM))

    h2 = jnp.maximum(
        jnp.dot(z_st, dw1_ref[...], preferred_element_type=jnp.float32)
        + db1_ref[...], 0.0)
    xr_ref[...] = (jnp.dot(h2, dw2_ref[...], preferred_element_type=jnp.float32)
                   + db2_ref[...])


@functools.cache
def _sc_gather_call():
    @functools.partial(
        pl.kernel,
        mesh=plsc.VectorSubcoreMesh(core_axis_name="c", subcore_axis_name="s",
                                    num_cores=SC_NC),
        out_type=jax.ShapeDtypeStruct((B, GATHER_D), jnp.float32),
        scratch_types=[
            pltpu.VMEM((B_PER_W,), jnp.int32),
            pltpu.VMEM((B_PER_W, GATHER_D), jnp.float32),
            pltpu.SemaphoreType.DMA,
        ],
    )
    def _sc_gather(table_hbm, idx_hbm, out_hbm, idx_v, rows_v, sem):
        base = lax.axis_index("s") * B_PER_W
        pltpu.sync_copy(idx_hbm.at[pl.ds(base, B_PER_W)], idx_v)
        pltpu.async_copy(table_hbm.at[idx_v], rows_v, sem).wait()
        pltpu.sync_copy(rows_v, out_hbm.at[pl.ds(base, B_PER_W)])

    return _sc_gather


_fused_call = pl.pallas_call(
    _fused_body,
    grid=(NB,),
    in_specs=[
        pl.BlockSpec((BM, INPUT_DIM), lambda i: (i, 0)),
        pl.BlockSpec((INPUT_DIM, HIDDEN_DIM), lambda i: (0, 0)),
        pl.BlockSpec((1, HIDDEN_DIM), lambda i: (0, 0)),
        pl.BlockSpec((HIDDEN_DIM, LATENT_DIM), lambda i: (0, 0)),
        pl.BlockSpec((1, LATENT_DIM), lambda i: (0, 0)),
        pl.BlockSpec((LATENT_DIM, NUM_EMBEDDINGS), lambda i: (0, 0)),
        pl.BlockSpec((NUM_EMBEDDINGS, LATENT_DIM), lambda i: (0, 0)),
        pl.BlockSpec((LATENT_DIM, HIDDEN_DIM), lambda i: (0, 0)),
        pl.BlockSpec((1, HIDDEN_DIM), lambda i: (0, 0)),
        pl.BlockSpec((HIDDEN_DIM, INPUT_DIM), lambda i: (0, 0)),
        pl.BlockSpec((1, INPUT_DIM), lambda i: (0, 0)),
    ],
    out_specs=[
        pl.BlockSpec((BM, LATENT_DIM), lambda i: (i, 0)),
        pl.BlockSpec((1, BM, 1), lambda i: (i, 0, 0)),
        pl.BlockSpec((BM, INPUT_DIM), lambda i: (i, 0)),
        pl.BlockSpec((1, 1), lambda i: (0, 0)),
    ],
    out_shape=[
        jax.ShapeDtypeStruct((B, LATENT_DIM), jnp.float32),
        jax.ShapeDtypeStruct((NB, BM, 1), jnp.int32),
        jax.ShapeDtypeStruct((B, INPUT_DIM), jnp.float32),
        jax.ShapeDtypeStruct((1, 1), jnp.float32),
    ],
)


def kernel(x, enc_W1, enc_b1, enc_W2, enc_b2, codebook,
           dec_W1, dec_b1, dec_W2, dec_b2):
    z_e, idx3, x_recon, loss = _fused_call(
        x, enc_W1, enc_b1.reshape(1, -1), enc_W2, enc_b2.reshape(1, -1),
        codebook.T, codebook, dec_W1, dec_b1.reshape(1, -1), dec_W2,
        dec_b2.reshape(1, -1))
    indices = idx3.reshape(B)
    codebook_pad = jnp.pad(codebook, ((0, 0), (0, GATHER_D - LATENT_DIM)))
    z_q_pad = _sc_gather_call()(codebook_pad, indices)
    z_q_st = z_q_pad[:, :LATENT_DIM]
    return (x_recon, z_e, z_q_st, indices, loss.reshape(()))


# speedup vs baseline: 1.8062x; 1.8062x over previous
import functools

import jax
import jax.numpy as jnp
from jax import lax
from jax.experimental import pallas as pl
from jax.experimental.pallas import tpu as pltpu
from jax.experimental.pallas import tpu_sc as plsc

B = 4096
INPUT_DIM = 768
HIDDEN_DIM = 512
LATENT_DIM = 32
NUM_EMBEDDINGS = 1024
BETA = 0.25

BM = 512
NB = B // BM

SC_NC = 2
SC_NS = 16
SC_NW = SC_NC * SC_NS
SC_L = 16
B_PER_W = B // SC_NW


def _encode_body(x_ref, w1_ref, b1_ref, w2_ref, b2_ref, cbt_ref,
                 ze_ref, idx_ref):
    h = jnp.maximum(
        jnp.dot(x_ref[...], w1_ref[...], preferred_element_type=jnp.float32)
        + b1_ref[...], 0.0)
    z_e = (jnp.dot(h, w2_ref[...], preferred_element_type=jnp.float32)
           + b2_ref[...])
    ze_ref[...] = z_e
    cbt = cbt_ref[...]
    cnorm2 = jnp.sum(cbt * cbt, axis=0, keepdims=True)
    scores = jnp.dot(z_e, cbt, preferred_element_type=jnp.float32,
                     precision=lax.Precision.HIGHEST)
    d2 = cnorm2 - 2.0 * scores
    dmin = jnp.min(d2, axis=1, keepdims=True)
    iota = lax.broadcasted_iota(jnp.int32, d2.shape, 1)
    cand = jnp.where(d2 == dmin, iota, NUM_EMBEDDINGS)
    idx_ref[0] = jnp.min(cand, axis=1, keepdims=True)


def _decode_body(ze_ref, zq_ref, w1_ref, b1_ref, w2_ref, b2_ref,
                 xr_ref, zst_ref, loss_ref):
    z_e = ze_ref[...]
    z_q = zq_ref[...]
    z_st = z_e + (z_q - z_e)
    zst_ref[...] = z_st
    diff = z_q - z_e
    part = jnp.sum(diff * diff, keepdims=True)

    @pl.when(pl.program_id(0) == 0)
    def _():
        loss_ref[...] = jnp.zeros_like(loss_ref)

    loss_ref[...] += part * ((1.0 + BETA) / (B * LATENT_DIM))
    h = jnp.maximum(
        jnp.dot(z_st, w1_ref[...], preferred_element_type=jnp.float32)
        + b1_ref[...], 0.0)
    xr_ref[...] = (jnp.dot(h, w2_ref[...], preferred_element_type=jnp.float32)
                   + b2_ref[...])


GATHER_D = 128


@functools.cache
def _sc_gather_call():
    @functools.partial(
        pl.kernel,
        mesh=plsc.VectorSubcoreMesh(core_axis_name="c", subcore_axis_name="s",
                                    num_cores=SC_NC),
        out_type=jax.ShapeDtypeStruct((B, GATHER_D), jnp.float32),
        scratch_types=[
            pltpu.VMEM_SHARED((NUM_EMBEDDINGS, GATHER_D), jnp.float32),
            pltpu.VMEM((B_PER_W,), jnp.int32),
            pltpu.VMEM((B_PER_W, GATHER_D), jnp.float32),
            pltpu.SemaphoreType.DMA,
        ],
    )
    def _sc_gather(table_hbm, idx_hbm, out_hbm, table_s, idx_v, rows_v, sem):
        wid = lax.axis_index("s") * SC_NC + lax.axis_index("c")
        base = wid * B_PER_W

        @pl.when(lax.axis_index("s") == 0)
        def _():
            pltpu.sync_copy(table_hbm, table_s)

        plsc.subcore_barrier()
        pltpu.sync_copy(idx_hbm.at[pl.ds(base, B_PER_W)], idx_v)
        pltpu.async_copy(table_s.at[idx_v], rows_v, sem).wait()
        pltpu.sync_copy(rows_v, out_hbm.at[pl.ds(base, B_PER_W)])

    return _sc_gather


_encode_call = pl.pallas_call(
    _encode_body,
    grid=(NB,),
    in_specs=[
        pl.BlockSpec((BM, INPUT_DIM), lambda i: (i, 0)),
        pl.BlockSpec((INPUT_DIM, HIDDEN_DIM), lambda i: (0, 0)),
        pl.BlockSpec((1, HIDDEN_DIM), lambda i: (0, 0)),
        pl.BlockSpec((HIDDEN_DIM, LATENT_DIM), lambda i: (0, 0)),
        pl.BlockSpec((1, LATENT_DIM), lambda i: (0, 0)),
        pl.BlockSpec((LATENT_DIM, NUM_EMBEDDINGS), lambda i: (0, 0)),
    ],
    out_specs=[
        pl.BlockSpec((BM, LATENT_DIM), lambda i: (i, 0)),
        pl.BlockSpec((1, BM, 1), lambda i: (i, 0, 0)),
    ],
    out_shape=[
        jax.ShapeDtypeStruct((B, LATENT_DIM), jnp.float32),
        jax.ShapeDtypeStruct((NB, BM, 1), jnp.int32),
    ],
)

_decode_call = pl.pallas_call(
    _decode_body,
    grid=(NB,),
    in_specs=[
        pl.BlockSpec((BM, LATENT_DIM), lambda i: (i, 0)),
        pl.BlockSpec((BM, LATENT_DIM), lambda i: (i, 0)),
        pl.BlockSpec((LATENT_DIM, HIDDEN_DIM), lambda i: (0, 0)),
        pl.BlockSpec((1, HIDDEN_DIM), lambda i: (0, 0)),
        pl.BlockSpec((HIDDEN_DIM, INPUT_DIM), lambda i: (0, 0)),
        pl.BlockSpec((1, INPUT_DIM), lambda i: (0, 0)),
    ],
    out_specs=[
        pl.BlockSpec((BM, INPUT_DIM), lambda i: (i, 0)),
        pl.BlockSpec((BM, LATENT_DIM), lambda i: (i, 0)),
        pl.BlockSpec((1, 1), lambda i: (0, 0)),
    ],
    out_shape=[
        jax.ShapeDtypeStruct((B, INPUT_DIM), jnp.float32),
        jax.ShapeDtypeStruct((B, LATENT_DIM), jnp.float32),
        jax.ShapeDtypeStruct((1, 1), jnp.float32),
    ],
)


def kernel(x, enc_W1, enc_b1, enc_W2, enc_b2, codebook,
           dec_W1, dec_b1, dec_W2, dec_b2):
    z_e, idx3 = _encode_call(
        x, enc_W1, enc_b1.reshape(1, -1), enc_W2, enc_b2.reshape(1, -1),
        codebook.T)
    indices = idx3.reshape(B)
    codebook_pad = jnp.pad(codebook, ((0, 0), (0, GATHER_D - LATENT_DIM)))
    z_q = _sc_gather_call()(codebook_pad, indices)[:, :LATENT_DIM]
    x_recon, z_q_st, loss = _decode_call(
        z_e, z_q, dec_W1, dec_b1.reshape(1, -1), dec_W2,
        dec_b2.reshape(1, -1))
    return (x_recon, z_e, z_q_st, indices, loss.reshape(()))


# speedup vs baseline: 1.8095x; 1.0018x over previous
import functools

import jax
import jax.numpy as jnp
from jax import lax
from jax.experimental import pallas as pl
from jax.experimental.pallas import tpu as pltpu
from jax.experimental.pallas import tpu_sc as plsc

B = 4096
INPUT_DIM = 768
HIDDEN_DIM = 512
LATENT_DIM = 32
NUM_EMBEDDINGS = 1024
BETA = 0.25

BM = 512
NB = B // BM

SC_NC = 2
SC_NS = 16
SC_NW = SC_NC * SC_NS
SC_L = 16
B_PER_W = B // SC_NW


def _encode_body(x_ref, w1_ref, b1_ref, w2_ref, b2_ref, cbt_ref,
                 ze_ref, idx_ref):
    h = jnp.maximum(
        jnp.dot(x_ref[...], w1_ref[...], preferred_element_type=jnp.float32)
        + b1_ref[...], 0.0)
    z_e = (jnp.dot(h, w2_ref[...], preferred_element_type=jnp.float32)
           + b2_ref[...])
    ze_ref[...] = z_e
    cbt = cbt_ref[...]
    cnorm2 = jnp.sum(cbt * cbt, axis=0, keepdims=True)
    scores = jnp.dot(z_e, cbt, preferred_element_type=jnp.float32,
                     precision=lax.Precision.HIGHEST)
    d2 = cnorm2 - 2.0 * scores
    dmin = jnp.min(d2, axis=1, keepdims=True)
    iota = lax.broadcasted_iota(jnp.int32, d2.shape, 1)
    cand = jnp.where(d2 == dmin, iota, NUM_EMBEDDINGS)
    idx_ref[0] = jnp.min(cand, axis=1, keepdims=True)


def _decode_body(ze_ref, zq_ref, w1_ref, b1_ref, w2_ref, b2_ref,
                 xr_ref, zst_ref, loss_ref):
    z_e = ze_ref[...]
    z_q = zq_ref[:, :LATENT_DIM]
    z_st = z_e + (z_q - z_e)
    zst_ref[...] = z_st
    diff = z_q - z_e
    part = jnp.sum(diff * diff, keepdims=True)

    @pl.when(pl.program_id(0) == 0)
    def _():
        loss_ref[...] = jnp.zeros_like(loss_ref)

    loss_ref[...] += part * ((1.0 + BETA) / (B * LATENT_DIM))
    h = jnp.maximum(
        jnp.dot(z_st, w1_ref[...], preferred_element_type=jnp.float32)
        + b1_ref[...], 0.0)
    xr_ref[...] = (jnp.dot(h, w2_ref[...], preferred_element_type=jnp.float32)
                   + b2_ref[...])


GATHER_D = 128


@functools.cache
def _sc_gather_call():
    @functools.partial(
        pl.kernel,
        mesh=plsc.VectorSubcoreMesh(core_axis_name="c", subcore_axis_name="s",
                                    num_cores=SC_NC),
        out_type=jax.ShapeDtypeStruct((B, GATHER_D), jnp.float32),
        scratch_types=[
            pltpu.VMEM_SHARED((NUM_EMBEDDINGS, GATHER_D), jnp.float32),
            pltpu.VMEM((B_PER_W,), jnp.int32),
            pltpu.VMEM((B_PER_W, GATHER_D), jnp.float32),
            pltpu.SemaphoreType.DMA,
        ],
    )
    def _sc_gather(table_hbm, idx_hbm, out_hbm, table_s, idx_v, rows_v, sem):
        wid = lax.axis_index("s") * SC_NC + lax.axis_index("c")
        base = wid * B_PER_W

        @pl.when(lax.axis_index("s") == 0)
        def _():
            pltpu.sync_copy(table_hbm, table_s)

        plsc.subcore_barrier()
        pltpu.sync_copy(idx_hbm.at[pl.ds(base, B_PER_W)], idx_v)
        pltpu.async_copy(table_s.at[idx_v], rows_v, sem).wait()
        pltpu.sync_copy(rows_v, out_hbm.at[pl.ds(base, B_PER_W)])

    return _sc_gather


_encode_call = pl.pallas_call(
    _encode_body,
    grid=(NB,),
    in_specs=[
        pl.BlockSpec((BM, INPUT_DIM), lambda i: (i, 0)),
        pl.BlockSpec((INPUT_DIM, HIDDEN_DIM), lambda i: (0, 0)),
        pl.BlockSpec((1, HIDDEN_DIM), lambda i: (0, 0)),
        pl.BlockSpec((HIDDEN_DIM, LATENT_DIM), lambda i: (0, 0)),
        pl.BlockSpec((1, LATENT_DIM), lambda i: (0, 0)),
        pl.BlockSpec((LATENT_DIM, NUM_EMBEDDINGS), lambda i: (0, 0)),
    ],
    out_specs=[
        pl.BlockSpec((BM, LATENT_DIM), lambda i: (i, 0)),
        pl.BlockSpec((1, BM, 1), lambda i: (i, 0, 0)),
    ],
    out_shape=[
        jax.ShapeDtypeStruct((B, LATENT_DIM), jnp.float32),
        jax.ShapeDtypeStruct((NB, BM, 1), jnp.int32),
    ],
)

_decode_call = pl.pallas_call(
    _decode_body,
    grid=(NB,),
    in_specs=[
        pl.BlockSpec((BM, LATENT_DIM), lambda i: (i, 0)),
        pl.BlockSpec((BM, GATHER_D), lambda i: (i, 0)),
        pl.BlockSpec((LATENT_DIM, HIDDEN_DIM), lambda i: (0, 0)),
        pl.BlockSpec((1, HIDDEN_DIM), lambda i: (0, 0)),
        pl.BlockSpec((HIDDEN_DIM, INPUT_DIM), lambda i: (0, 0)),
        pl.BlockSpec((1, INPUT_DIM), lambda i: (0, 0)),
    ],
    out_specs=[
        pl.BlockSpec((BM, INPUT_DIM), lambda i: (i, 0)),
        pl.BlockSpec((BM, LATENT_DIM), lambda i: (i, 0)),
        pl.BlockSpec((1, 1), lambda i: (0, 0)),
    ],
    out_shape=[
        jax.ShapeDtypeStruct((B, INPUT_DIM), jnp.float32),
        jax.ShapeDtypeStruct((B, LATENT_DIM), jnp.float32),
        jax.ShapeDtypeStruct((1, 1), jnp.float32),
    ],
)


def kernel(x, enc_W1, enc_b1, enc_W2, enc_b2, codebook,
           dec_W1, dec_b1, dec_W2, dec_b2):
    z_e, idx3 = _encode_call(
        x, enc_W1, enc_b1.reshape(1, -1), enc_W2, enc_b2.reshape(1, -1),
        codebook.T)
    indices = idx3.reshape(B)
    codebook_pad = jnp.pad(codebook, ((0, 0), (0, GATHER_D - LATENT_DIM)))
    z_q = _sc_gather_call()(codebook_pad, indices)
    x_recon, z_q_st, loss = _decode_call(
        z_e, z_q, dec_W1, dec_b1.reshape(1, -1), dec_W2,
        dec_b2.reshape(1, -1))
    return (x_recon, z_e, z_q_st, indices, loss.reshape(()))


# speedup vs baseline: 1.9005x; 1.0503x over previous
import functools

import jax
import jax.numpy as jnp
from jax import lax
from jax.experimental import pallas as pl
from jax.experimental.pallas import tpu as pltpu
from jax.experimental.pallas import tpu_sc as plsc

B = 4096
INPUT_DIM = 768
HIDDEN_DIM = 512
LATENT_DIM = 32
NUM_EMBEDDINGS = 1024
BETA = 0.25

BM = 1024
NB = B // BM

SC_NC = 2
SC_NS = 16
SC_NW = SC_NC * SC_NS
SC_L = 16
B_PER_W = B // SC_NW


def _encode_body(x_ref, w1_ref, b1_ref, w2_ref, b2_ref, cbt_ref,
                 ze_ref, idx_ref):
    h = jnp.maximum(
        jnp.dot(x_ref[...], w1_ref[...], preferred_element_type=jnp.float32)
        + b1_ref[...], 0.0)
    z_e = (jnp.dot(h, w2_ref[...], preferred_element_type=jnp.float32)
           + b2_ref[...])
    ze_ref[...] = z_e
    cbt = cbt_ref[...]
    cnorm2 = jnp.sum(cbt * cbt, axis=0, keepdims=True)
    scores = jnp.dot(z_e, cbt, preferred_element_type=jnp.float32,
                     precision=lax.Precision.HIGHEST)
    d2 = cnorm2 - 2.0 * scores
    dmin = jnp.min(d2, axis=1, keepdims=True)
    iota = lax.broadcasted_iota(jnp.int32, d2.shape, 1)
    cand = jnp.where(d2 == dmin, iota, NUM_EMBEDDINGS)
    idx_ref[0] = jnp.min(cand, axis=1, keepdims=True)


def _decode_body(ze_ref, zq_ref, w1_ref, b1_ref, w2_ref, b2_ref,
                 xr_ref, zst_ref, loss_ref):
    z_e = ze_ref[...]
    z_q = zq_ref[:, :LATENT_DIM]
    z_st = z_e + (z_q - z_e)
    zst_ref[...] = z_st
    diff = z_q - z_e
    part = jnp.sum(diff * diff, keepdims=True)

    @pl.when(pl.program_id(0) == 0)
    def _():
        loss_ref[...] = jnp.zeros_like(loss_ref)

    loss_ref[...] += part * ((1.0 + BETA) / (B * LATENT_DIM))
    h = jnp.maximum(
        jnp.dot(z_st, w1_ref[...], preferred_element_type=jnp.float32)
        + b1_ref[...], 0.0)
    xr_ref[...] = (jnp.dot(h, w2_ref[...], preferred_element_type=jnp.float32)
                   + b2_ref[...])


GATHER_D = 128


@functools.cache
def _sc_gather_call():
    @functools.partial(
        pl.kernel,
        mesh=plsc.VectorSubcoreMesh(core_axis_name="c", subcore_axis_name="s",
                                    num_cores=SC_NC),
        out_type=jax.ShapeDtypeStruct((B, GATHER_D), jnp.float32),
        scratch_types=[
            pltpu.VMEM_SHARED((NUM_EMBEDDINGS, GATHER_D), jnp.float32),
            pltpu.VMEM((B_PER_W,), jnp.int32),
            pltpu.VMEM((B_PER_W, GATHER_D), jnp.float32),
            pltpu.SemaphoreType.DMA,
        ],
    )
    def _sc_gather(table_hbm, idx_hbm, out_hbm, table_s, idx_v, rows_v, sem):
        wid = lax.axis_index("s") * SC_NC + lax.axis_index("c")
        base = wid * B_PER_W

        @pl.when(lax.axis_index("s") == 0)
        def _():
            pltpu.sync_copy(table_hbm, table_s)

        plsc.subcore_barrier()
        pltpu.sync_copy(idx_hbm.at[pl.ds(base, B_PER_W)], idx_v)
        pltpu.async_copy(table_s.at[idx_v], rows_v, sem).wait()
        pltpu.sync_copy(rows_v, out_hbm.at[pl.ds(base, B_PER_W)])

    return _sc_gather


_encode_call = pl.pallas_call(
    _encode_body,
    grid=(NB,),
    in_specs=[
        pl.BlockSpec((BM, INPUT_DIM), lambda i: (i, 0)),
        pl.BlockSpec((INPUT_DIM, HIDDEN_DIM), lambda i: (0, 0)),
        pl.BlockSpec((1, HIDDEN_DIM), lambda i: (0, 0)),
        pl.BlockSpec((HIDDEN_DIM, LATENT_DIM), lambda i: (0, 0)),
        pl.BlockSpec((1, LATENT_DIM), lambda i: (0, 0)),
        pl.BlockSpec((LATENT_DIM, NUM_EMBEDDINGS), lambda i: (0, 0)),
    ],
    out_specs=[
        pl.BlockSpec((BM, LATENT_DIM), lambda i: (i, 0)),
        pl.BlockSpec((1, BM, 1), lambda i: (i, 0, 0)),
    ],
    out_shape=[
        jax.ShapeDtypeStruct((B, LATENT_DIM), jnp.float32),
        jax.ShapeDtypeStruct((NB, BM, 1), jnp.int32),
    ],
)

_decode_call = pl.pallas_call(
    _decode_body,
    grid=(NB,),
    in_specs=[
        pl.BlockSpec((BM, LATENT_DIM), lambda i: (i, 0)),
        pl.BlockSpec((BM, GATHER_D), lambda i: (i, 0)),
        pl.BlockSpec((LATENT_DIM, HIDDEN_DIM), lambda i: (0, 0)),
        pl.BlockSpec((1, HIDDEN_DIM), lambda i: (0, 0)),
        pl.BlockSpec((HIDDEN_DIM, INPUT_DIM), lambda i: (0, 0)),
        pl.BlockSpec((1, INPUT_DIM), lambda i: (0, 0)),
    ],
    out_specs=[
        pl.BlockSpec((BM, INPUT_DIM), lambda i: (i, 0)),
        pl.BlockSpec((BM, LATENT_DIM), lambda i: (i, 0)),
        pl.BlockSpec((1, 1), lambda i: (0, 0)),
    ],
    out_shape=[
        jax.ShapeDtypeStruct((B, INPUT_DIM), jnp.float32),
        jax.ShapeDtypeStruct((B, LATENT_DIM), jnp.float32),
        jax.ShapeDtypeStruct((1, 1), jnp.float32),
    ],
)


def kernel(x, enc_W1, enc_b1, enc_W2, enc_b2, codebook,
           dec_W1, dec_b1, dec_W2, dec_b2):
    z_e, idx3 = _encode_call(
        x, enc_W1, enc_b1.reshape(1, -1), enc_W2, enc_b2.reshape(1, -1),
        codebook.T)
    indices = idx3.reshape(B)
    codebook_pad = jnp.pad(codebook, ((0, 0), (0, GATHER_D - LATENT_DIM)))
    z_q = _sc_gather_call()(codebook_pad, indices)
    x_recon, z_q_st, loss = _decode_call(
        z_e, z_q, dec_W1, dec_b1.reshape(1, -1), dec_W2,
        dec_b2.reshape(1, -1))
    return (x_recon, z_e, z_q_st, indices, loss.reshape(()))
